# Initial kernel scaffold; baseline (speedup 1.0000x reference)
import functools

import jax
import jax.numpy as jnp
from jax import lax
from jax.experimental import pallas as pl
from jax.experimental.pallas import tpu as pltpu
from jax.experimental.pallas import tpu_sc as plsc

NU, NI, NE, NA, CH = 8192, 5000, 10000, 128, 256
NEDGE, NNZ_UA, NNZ_IA = 160000, 65536, 40000
NREL = 8
NW = 32
GCH = 4
NGRP = CH // GCH
NPASS = NGRP // NW
ECHUNK = 4000
UCHUNK = 4096
ICHUNK = 4000
EPS = 1e-12

_mesh = plsc.VectorSubcoreMesh(core_axis_name="c", subcore_axis_name="s")


def _cvec(c):
    return jnp.full((16,), c, jnp.int32)


def _kg_phase(tbl, w_t, head, tail, et, kg_out, src, acc, wsl, b1, b2, b3,
              cnt=None, cntp=None):
    wid = lax.axis_index("s") * 2 + lax.axis_index("c")
    zeros16 = jnp.zeros((16,), jnp.float32)
    ones16 = jnp.ones((16,), jnp.float32)
    z16i = jnp.zeros((16,), jnp.int32)

    if cnt is not None:
        def zc(i, carry):
            cnt[0, pl.ds(i * 16, 16)] = zeros16
            return carry
        lax.fori_loop(0, NE // 16, zc, 0)

    for p in range(NPASS):
        g = wid * NPASS + p
        pltpu.sync_copy(tbl.at[pl.ds(g * GCH, GCH)], src)
        pltpu.sync_copy(w_t.at[pl.ds(g * GCH, GCH)], wsl)

        def za(i, carry):
            for c in range(GCH):
                acc[c, pl.ds(i * 16, 16)] = zeros16
            return carry
        lax.fori_loop(0, NE // 16, za, 0)

        def chunk_body(k, carry):
            off = k * ECHUNK
            pltpu.sync_copy(tail.at[pl.ds(off, ECHUNK)], b1.at[pl.ds(0, ECHUNK)])
            pltpu.sync_copy(head.at[pl.ds(off, ECHUNK)], b2.at[pl.ds(0, ECHUNK)])
            pltpu.sync_copy(et.at[pl.ds(off, ECHUNK)], b3.at[pl.ds(0, ECHUNK)])
            do_cnt = lax.rem(k, NW) == wid
            mvec = jnp.broadcast_to(do_cnt, (16,))

            def grp(j, carry2):
                t16 = b1[pl.ds(j * 16, 16)]
                h16 = b2[pl.ds(j * 16, 16)]
                e16 = b3[pl.ds(j * 16, 16)]
                for c in range(GCH):
                    sv = plsc.load_gather(src, [_cvec(c), t16])
                    rv = plsc.load_gather(wsl, [_cvec(c), e16])
                    plsc.addupdate_scatter(acc, [_cvec(c), h16], sv * rv)
                if cnt is not None and p == 0:
                    plsc.addupdate_scatter(cnt, [z16i, h16], ones16, mask=mvec)
                return carry2
            lax.fori_loop(0, ECHUNK // 16, grp, 0)
            return carry
        lax.fori_loop(0, NEDGE // ECHUNK, chunk_body, 0)

        pltpu.sync_copy(acc, kg_out.at[pl.ds(g * GCH, GCH)])
    if cnt is not None:
        pltpu.sync_copy(cnt, cntp.at[pl.ds(wid, 1)])


def _spmm_phase(asp_t, rows, cols, vals, out, n_rows, nnz, chunk,
                acc, asl, b1, b2, bv):
    wid = lax.axis_index("s") * 2 + lax.axis_index("c")
    zeros16 = jnp.zeros((16,), jnp.float32)
    nz16 = (n_rows + 15) // 16

    for p in range(NPASS):
        g = wid * NPASS + p
        pltpu.sync_copy(asp_t.at[pl.ds(g * GCH, GCH)], asl)

        def za(i, carry):
            for c in range(GCH):
                acc[c, pl.ds(i * 16, 16)] = zeros16
            return carry
        lax.fori_loop(0, nz16, za, 0)

        def chunk_body(k, carry):
            off = k * chunk
            pltpu.sync_copy(rows.at[pl.ds(off, chunk)], b1.at[pl.ds(0, chunk)])
            pltpu.sync_copy(cols.at[pl.ds(off, chunk)], b2.at[pl.ds(0, chunk)])
            pltpu.sync_copy(vals.at[pl.ds(off, chunk)], bv.at[pl.ds(0, chunk)])

            def grp(j, carry2):
                r16 = b1[pl.ds(j * 16, 16)]
                c16 = b2[pl.ds(j * 16, 16)]
                v16 = bv[pl.ds(j * 16, 16)]
                for c in range(GCH):
                    av = plsc.load_gather(asl, [_cvec(c), c16])
                    plsc.addupdate_scatter(acc, [_cvec(c), r16], av * v16)
                return carry2
            lax.fori_loop(0, chunk // 16, grp, 0)
            return carry
        lax.fori_loop(0, nnz // chunk, chunk_body, 0)

        for c in range(GCH):
            pltpu.sync_copy(acc.at[pl.ds(c, 1), pl.ds(0, n_rows)],
                            out.at[pl.ds(g * GCH + c, 1)])


def _sc1_body(ent_t, w_t, asp_t, head, tail, et,
              ua_r, ua_c, ua_v, ia_r, ia_c, ia_v,
              kg, cntp, uas, ias,
              src, acc, cnt, wsl, asl, b1, b2, b3, bv):
    _kg_phase(ent_t, w_t, head, tail, et, kg, src, acc, wsl, b1, b2, b3,
              cnt=cnt, cntp=cntp)
    _spmm_phase(asp_t, ua_r, ua_c, ua_v, uas, NU, NNZ_UA, UCHUNK,
                acc, asl, b1, b2, bv)
    _spmm_phase(asp_t, ia_r, ia_c, ia_v, ias, NI, NNZ_IA, ICHUNK,
                acc, asl, b1, b2, bv)


def _sc2_body(ent_t, w_t, head, tail, et, kg,
              src, acc, wsl, b1, b2, b3):
    _kg_phase(ent_t, w_t, head, tail, et, kg, src, acc, wsl, b1, b2, b3)


_f32 = jnp.float32
_sc1 = functools.partial(
    pl.kernel, _sc1_body,
    out_type=(jax.ShapeDtypeStruct((CH, NE), _f32),
              jax.ShapeDtypeStruct((NW, NE), _f32),
              jax.ShapeDtypeStruct((CH, NU), _f32),
              jax.ShapeDtypeStruct((CH, NI), _f32)),
    mesh=_mesh,
    scratch_types=(pltpu.VMEM((GCH, NE), _f32),
                   pltpu.VMEM((GCH, NE), _f32),
                   pltpu.VMEM((1, NE), _f32),
                   pltpu.VMEM((GCH, NREL), _f32),
                   pltpu.VMEM((GCH, NA), _f32),
                   pltpu.VMEM((UCHUNK,), jnp.int32),
                   pltpu.VMEM((UCHUNK,), jnp.int32),
                   pltpu.VMEM((UCHUNK,), jnp.int32),
                   pltpu.VMEM((UCHUNK,), _f32)))()

_sc2 = functools.partial(
    pl.kernel, _sc2_body,
    out_type=jax.ShapeDtypeStruct((CH, NE), _f32),
    mesh=_mesh,
    scratch_types=(pltpu.VMEM((GCH, NE), _f32),
                   pltpu.VMEM((GCH, NE), _f32),
                   pltpu.VMEM((GCH, NREL), _f32),
                   pltpu.VMEM((UCHUNK,), jnp.int32),
                   pltpu.VMEM((UCHUNK,), jnp.int32),
                   pltpu.VMEM((UCHUNK,), jnp.int32)))()


def _tc1_body(kg_ref, cntp_ref, e1_ref, inv_ref):
    cnt = jnp.sum(cntp_ref[...], axis=0, keepdims=True)
    inv = 1.0 / jnp.maximum(cnt, 1.0)
    agg = kg_ref[...] * inv
    ss = jnp.sum(agg * agg, axis=0, keepdims=True)
    rn = 1.0 / jnp.maximum(jnp.sqrt(ss), EPS)
    e1_ref[...] = agg * rn
    inv_ref[...] = inv


def _tc1(kg, cntp):
    b = 1000
    return pl.pallas_call(
        _tc1_body,
        grid=(NE // b,),
        in_specs=[pl.BlockSpec((CH, b), lambda i: (0, i)),
                  pl.BlockSpec((NW, b), lambda i: (0, i))],
        out_specs=[pl.BlockSpec((CH, b), lambda i: (0, i)),
                   pl.BlockSpec((1, b), lambda i: (0, i))],
        out_shape=[jax.ShapeDtypeStruct((CH, NE), _f32),
                   jax.ShapeDtypeStruct((1, NE), _f32)],
    )(kg, cntp)


def _tc_item_body(kg2_ref, inv_ref, e1_ref, ent0_ref, ias_ref, item_ref,
                  w1_ref, w2_ref, out_ref):
    agg2 = kg2_ref[...] * inv_ref[...]
    ss2 = jnp.sum(agg2 * agg2, axis=0, keepdims=True)
    e2 = agg2 * (1.0 / jnp.maximum(jnp.sqrt(ss2), EPS))
    ia2 = 2.0 * ias_ref[...]
    ssi = jnp.sum(ia2 * ia2, axis=0, keepdims=True)
    i_n = ia2 * (1.0 / jnp.maximum(jnp.sqrt(ssi), EPS))
    ent_res = ent0_ref[...] + jnp.transpose(e1_ref[...]) + jnp.transpose(e2)
    item_pre = item_ref[...] + 2.0 * jnp.transpose(i_n)
    out_ref[...] = w1_ref[0, 0] * ent_res + w2_ref[0, 0] * item_pre


def _tc_item(kg2, inv, e1, ent0, ias, item_emb, w1, w2):
    b = 500
    return pl.pallas_call(
        _tc_item_body,
        grid=(NI // b,),
        in_specs=[pl.BlockSpec((CH, b), lambda i: (0, i)),
                  pl.BlockSpec((1, b), lambda i: (0, i)),
                  pl.BlockSpec((CH, b), lambda i: (0, i)),
                  pl.BlockSpec((b, CH), lambda i: (i, 0)),
                  pl.BlockSpec((CH, b), lambda i: (0, i)),
                  pl.BlockSpec((b, CH), lambda i: (i, 0)),
                  pl.BlockSpec((1, 1), lambda i: (0, 0)),
                  pl.BlockSpec((1, 1), lambda i: (0, 0))],
        out_specs=pl.BlockSpec((b, CH), lambda i: (i, 0)),
        out_shape=jax.ShapeDtypeStruct((NI, CH), _f32),
    )(kg2, inv, e1, ent0, ias, item_emb, w1, w2)


def _tc_user_body(uas_ref, user_ref, out_ref):
    a = 2.0 * uas_ref[...]
    ss = jnp.sum(a * a, axis=0, keepdims=True)
    u = a * (1.0 / jnp.maximum(jnp.sqrt(ss), EPS))
    out_ref[...] = user_ref[...] + 2.0 * jnp.transpose(u)


def _tc_user(uas, user_emb):
    b = 512
    return pl.pallas_call(
        _tc_user_body,
        grid=(NU // b,),
        in_specs=[pl.BlockSpec((CH, b), lambda i: (0, i)),
                  pl.BlockSpec((b, CH), lambda i: (i, 0))],
        out_specs=pl.BlockSpec((b, CH), lambda i: (i, 0)),
        out_shape=jax.ShapeDtypeStruct((NU, CH), _f32),
    )(uas, user_emb)


def kernel(user_emb, item_emb, entity_emb, aspect_emb, weight, W1, W2,
           edge_index, edge_type, ua_indices, ua_values, ia_indices,
           ia_values):
    head = edge_index[0]
    tail = edge_index[1]
    et = edge_type - 2
    ent0_t = entity_emb.T
    w_t = weight.T
    asp_t = aspect_emb.T

    kg1, cntp, uas, ias = _sc1(ent0_t, w_t, asp_t, head, tail, et,
                               ua_indices[0], ua_indices[1], ua_values,
                               ia_indices[0], ia_indices[1], ia_values)
    e1_t, inv = _tc1(kg1, cntp)
    kg2 = _sc2(e1_t, w_t, head, tail, et)
    item_res = _tc_item(kg2, inv, e1_t, entity_emb[:NI], ias, item_emb,
                        W1.reshape(1, 1), W2.reshape(1, 1))
    user_res = _tc_user(uas, user_emb)
    return (item_res, user_res)


# speedup vs baseline: 1.3414x; 1.3414x over previous
import jax
import jax.numpy as jnp
from jax import lax
from jax.experimental import pallas as pl
from jax.experimental.pallas import tpu as pltpu
from jax.experimental.pallas import tpu_sc as plsc

NU, NI, NE, NA, CH = 8192, 5000, 10000, 128, 256
NEDGE, NNZ_UA, NNZ_IA = 160000, 65536, 40000
NREL = 8
NW = 32
GCH = 4
NGRP = CH // GCH
NPASS = NGRP // NW
ECHUNK = 4000
UCHUNK = 4096
ICHUNK = 4000
EPS = 1e-12

_f32 = jnp.float32
_mesh = plsc.VectorSubcoreMesh(core_axis_name="c", subcore_axis_name="s")


def _kg_phase(tbl, w_t, head, tail, et, kg_out, src, acc, wsl, b1, b2, b3,
              cnt=None, cntp=None):
    wid = lax.axis_index("s") * 2 + lax.axis_index("c")
    zeros16 = jnp.zeros((16,), _f32)
    ones16 = jnp.ones((16,), _f32)

    if cnt is not None:
        def zc(i, carry):
            cnt[pl.ds(i * 16, 16)] = zeros16
            return carry
        lax.fori_loop(0, NE // 16, zc, 0)

    for p in range(NPASS):
        g = wid * NPASS + p
        pltpu.sync_copy(tbl.at[pl.ds(g * (GCH * NE), GCH * NE)], src)
        pltpu.sync_copy(w_t.at[pl.ds(g * (GCH * NREL), GCH * NREL)], wsl)

        def za(i, carry):
            acc[pl.ds(i * 16, 16)] = zeros16
            return carry
        lax.fori_loop(0, GCH * NE // 16, za, 0)

        def chunk_body(k, carry):
            off = k * ECHUNK
            pltpu.sync_copy(tail.at[pl.ds(off, ECHUNK)], b1.at[pl.ds(0, ECHUNK)])
            pltpu.sync_copy(head.at[pl.ds(off, ECHUNK)], b2.at[pl.ds(0, ECHUNK)])
            pltpu.sync_copy(et.at[pl.ds(off, ECHUNK)], b3.at[pl.ds(0, ECHUNK)])
            mvec = jnp.broadcast_to(lax.rem(k, NW) == wid, (16,))

            def grp(j, carry2):
                t16 = b1[pl.ds(j * 16, 16)]
                h16 = b2[pl.ds(j * 16, 16)]
                e16 = b3[pl.ds(j * 16, 16)]
                for c in range(GCH):
                    sv = plsc.load_gather(src, [t16 + c * NE])
                    rv = plsc.load_gather(wsl, [e16 + c * NREL])
                    plsc.addupdate_scatter(acc, [h16 + c * NE], sv * rv)
                if cnt is not None and p == 0:
                    plsc.addupdate_scatter(cnt, [h16], ones16, mask=mvec)
                return carry2
            lax.fori_loop(0, ECHUNK // 16, grp, 0)
            return carry
        lax.fori_loop(0, NEDGE // ECHUNK, chunk_body, 0)

        pltpu.sync_copy(acc, kg_out.at[pl.ds(g * (GCH * NE), GCH * NE)])
    if cnt is not None:
        pltpu.sync_copy(cnt, cntp.at[pl.ds(wid * NE, NE)])


def _spmm_phase(asp_t, rows, cols, vals, out, n_rows, nnz, chunk,
                acc, asl, b1, b2, bv):
    wid = lax.axis_index("s") * 2 + lax.axis_index("c")
    zeros16 = jnp.zeros((16,), _f32)
    nz16 = (GCH * NE) // 16

    for p in range(NPASS):
        g = wid * NPASS + p
        pltpu.sync_copy(asp_t.at[pl.ds(g * (GCH * NA), GCH * NA)], asl)

        def za(i, carry):
            acc[pl.ds(i * 16, 16)] = zeros16
            return carry
        lax.fori_loop(0, nz16, za, 0)

        def chunk_body(k, carry):
            off = k * chunk
            pltpu.sync_copy(rows.at[pl.ds(off, chunk)], b1.at[pl.ds(0, chunk)])
            pltpu.sync_copy(cols.at[pl.ds(off, chunk)], b2.at[pl.ds(0, chunk)])
            pltpu.sync_copy(vals.at[pl.ds(off, chunk)], bv.at[pl.ds(0, chunk)])

            def grp(j, carry2):
                r16 = b1[pl.ds(j * 16, 16)]
                c16 = b2[pl.ds(j * 16, 16)]
                v16 = bv[pl.ds(j * 16, 16)]
                for c in range(GCH):
                    av = plsc.load_gather(asl, [c16 + c * NA])
                    plsc.addupdate_scatter(acc, [r16 + c * NE], av * v16)
                return carry2
            lax.fori_loop(0, chunk // 16, grp, 0)
            return carry
        lax.fori_loop(0, nnz // chunk, chunk_body, 0)

        for c in range(GCH):
            pltpu.sync_copy(acc.at[pl.ds(c * NE, n_rows)],
                            out.at[pl.ds((g * GCH + c) * n_rows, n_rows)])


def _sc1_body(ent_t, w_t, asp_t, head, tail, et,
              ua_r, ua_c, ua_v, ia_r, ia_c, ia_v,
              kg, cntp, uas, ias,
              src, acc, cnt, wsl, asl, b1, b2, b3, bv):
    _kg_phase(ent_t, w_t, head, tail, et, kg, src, acc, wsl, b1, b2, b3,
              cnt=cnt, cntp=cntp)
    _spmm_phase(asp_t, ua_r, ua_c, ua_v, uas, NU, NNZ_UA, UCHUNK,
                acc, asl, b1, b2, bv)
    _spmm_phase(asp_t, ia_r, ia_c, ia_v, ias, NI, NNZ_IA, ICHUNK,
                acc, asl, b1, b2, bv)


def _sc2_body(ent_t, w_t, head, tail, et, kg,
              src, acc, wsl, b1, b2, b3):
    _kg_phase(ent_t, w_t, head, tail, et, kg, src, acc, wsl, b1, b2, b3)


_sc1 = pl.kernel(
    _sc1_body,
    out_type=(jax.ShapeDtypeStruct((CH * NE,), _f32),
              jax.ShapeDtypeStruct((NW * NE,), _f32),
              jax.ShapeDtypeStruct((CH * NU,), _f32),
              jax.ShapeDtypeStruct((CH * NI,), _f32)),
    mesh=_mesh,
    compiler_params=pltpu.CompilerParams(needs_layout_passes=False),
    scratch_types=(pltpu.VMEM((GCH * NE,), _f32),
                   pltpu.VMEM((GCH * NE,), _f32),
                   pltpu.VMEM((NE,), _f32),
                   pltpu.VMEM((GCH * NREL,), _f32),
                   pltpu.VMEM((GCH * NA,), _f32),
                   pltpu.VMEM((UCHUNK,), jnp.int32),
                   pltpu.VMEM((UCHUNK,), jnp.int32),
                   pltpu.VMEM((UCHUNK,), jnp.int32),
                   pltpu.VMEM((UCHUNK,), _f32)))

_sc2 = pl.kernel(
    _sc2_body,
    out_type=jax.ShapeDtypeStruct((CH * NE,), _f32),
    mesh=_mesh,
    compiler_params=pltpu.CompilerParams(needs_layout_passes=False),
    scratch_types=(pltpu.VMEM((GCH * NE,), _f32),
                   pltpu.VMEM((GCH * NE,), _f32),
                   pltpu.VMEM((GCH * NREL,), _f32),
                   pltpu.VMEM((UCHUNK,), jnp.int32),
                   pltpu.VMEM((UCHUNK,), jnp.int32),
                   pltpu.VMEM((UCHUNK,), jnp.int32)))


def _tc1_body(kg_ref, cntp_ref, e1_ref, inv_ref):
    cnt = jnp.sum(cntp_ref[...], axis=0, keepdims=True)
    inv = 1.0 / jnp.maximum(cnt, 1.0)
    agg = kg_ref[...] * inv
    ss = jnp.sum(agg * agg, axis=0, keepdims=True)
    rn = 1.0 / jnp.maximum(jnp.sqrt(ss), EPS)
    e1_ref[...] = agg * rn
    inv_ref[...] = inv


def _tc1(kg, cntp):
    b = 1024
    return pl.pallas_call(
        _tc1_body,
        grid=(pl.cdiv(NE, b),),
        in_specs=[pl.BlockSpec((CH, b), lambda i: (0, i)),
                  pl.BlockSpec((NW, b), lambda i: (0, i))],
        out_specs=[pl.BlockSpec((CH, b), lambda i: (0, i)),
                   pl.BlockSpec((1, b), lambda i: (0, i))],
        out_shape=[jax.ShapeDtypeStruct((CH, NE), _f32),
                   jax.ShapeDtypeStruct((1, NE), _f32)],
    )(kg, cntp)


def _tc_item_body(kg2_ref, inv_ref, e1_ref, ent0_ref, ias_ref, item_ref,
                  w1_ref, w2_ref, out_ref):
    agg2 = kg2_ref[...] * inv_ref[...]
    ss2 = jnp.sum(agg2 * agg2, axis=0, keepdims=True)
    e2 = agg2 * (1.0 / jnp.maximum(jnp.sqrt(ss2), EPS))
    ia2 = 2.0 * ias_ref[...]
    ssi = jnp.sum(ia2 * ia2, axis=0, keepdims=True)
    i_n = ia2 * (1.0 / jnp.maximum(jnp.sqrt(ssi), EPS))
    ent_res = ent0_ref[...] + jnp.transpose(e1_ref[...]) + jnp.transpose(e2)
    item_pre = item_ref[...] + 2.0 * jnp.transpose(i_n)
    out_ref[...] = w1_ref[0, 0] * ent_res + w2_ref[0, 0] * item_pre


def _tc_item(kg2, inv, e1, ent0, ias, item_emb, w1, w2):
    b = 512
    return pl.pallas_call(
        _tc_item_body,
        grid=(pl.cdiv(NI, b),),
        in_specs=[pl.BlockSpec((CH, b), lambda i: (0, i)),
                  pl.BlockSpec((1, b), lambda i: (0, i)),
                  pl.BlockSpec((CH, b), lambda i: (0, i)),
                  pl.BlockSpec((b, CH), lambda i: (i, 0)),
                  pl.BlockSpec((CH, b), lambda i: (0, i)),
                  pl.BlockSpec((b, CH), lambda i: (i, 0)),
                  pl.BlockSpec((1, 1), lambda i: (0, 0)),
                  pl.BlockSpec((1, 1), lambda i: (0, 0))],
        out_specs=pl.BlockSpec((b, CH), lambda i: (i, 0)),
        out_shape=jax.ShapeDtypeStruct((NI, CH), _f32),
    )(kg2, inv, e1, ent0, ias, item_emb, w1, w2)


def _tc_user_body(uas_ref, user_ref, out_ref):
    a = 2.0 * uas_ref[...]
    ss = jnp.sum(a * a, axis=0, keepdims=True)
    u = a * (1.0 / jnp.maximum(jnp.sqrt(ss), EPS))
    out_ref[...] = user_ref[...] + 2.0 * jnp.transpose(u)


def _tc_user(uas, user_emb):
    b = 512
    return pl.pallas_call(
        _tc_user_body,
        grid=(NU // b,),
        in_specs=[pl.BlockSpec((CH, b), lambda i: (0, i)),
                  pl.BlockSpec((b, CH), lambda i: (i, 0))],
        out_specs=pl.BlockSpec((b, CH), lambda i: (i, 0)),
        out_shape=jax.ShapeDtypeStruct((NU, CH), _f32),
    )(uas, user_emb)


def kernel(user_emb, item_emb, entity_emb, aspect_emb, weight, W1, W2,
           edge_index, edge_type, ua_indices, ua_values, ia_indices,
           ia_values):
    head = edge_index[0]
    tail = edge_index[1]
    et = edge_type - 2
    ent0_t = entity_emb.T.reshape(-1)
    w_t = weight.T.reshape(-1)
    asp_t = aspect_emb.T.reshape(-1)

    kg1, cntp, uas, ias = _sc1(ent0_t, w_t, asp_t, head, tail, et,
                               ua_indices[0], ua_indices[1], ua_values,
                               ia_indices[0], ia_indices[1], ia_values)
    e1_t, inv = _tc1(kg1.reshape(CH, NE), cntp.reshape(NW, NE))
    kg2 = _sc2(e1_t.reshape(-1), w_t, head, tail, et)
    item_res = _tc_item(kg2.reshape(CH, NE), inv, e1_t, entity_emb[:NI],
                        ias.reshape(CH, NI), item_emb,
                        W1.reshape(1, 1), W2.reshape(1, 1))
    user_res = _tc_user(uas.reshape(CH, NU), user_emb)
    return (item_res, user_res)


# speedup vs baseline: 1.5782x; 1.1766x over previous
import jax
import jax.numpy as jnp
from jax import lax
from jax.experimental import pallas as pl
from jax.experimental.pallas import tpu as pltpu
from jax.experimental.pallas import tpu_sc as plsc

NU, NI, NE, NA, CH = 8192, 5000, 10000, 128, 256
NEDGE, NNZ_UA, NNZ_IA = 160000, 65536, 40000
NREL = 8
NW = 32
GCH = 4
NGRP = CH // GCH
NPASS = NGRP // NW
ECHUNK = 10000
UCHUNK = 4096
ICHUNK = 4000
EPS = 1e-12

_f32 = jnp.float32
_i32 = jnp.int32
_mesh = plsc.VectorSubcoreMesh(core_axis_name="c", subcore_axis_name="s")


def _zero(ref, n16):
    zeros16 = jnp.zeros((16,), _f32)

    def za(i, carry):
        ref[pl.ds(i * 16, 16)] = zeros16
        return carry
    lax.fori_loop(0, n16, za, 0)


def _stream_chunks(src_hbm, vals_hbm, nchunks, chunk, bpk, bv, sems, vsems,
                   process):
    def start(k, slot):
        pltpu.make_async_copy(src_hbm.at[pl.ds(k * chunk, chunk)],
                              bpk.at[pl.ds(slot * chunk, chunk)],
                              sems[slot]).start()
        if vals_hbm is not None:
            pltpu.make_async_copy(vals_hbm.at[pl.ds(k * chunk, chunk)],
                                  bv.at[pl.ds(slot * chunk, chunk)],
                                  vsems[slot]).start()

    def wait(slot):
        pltpu.make_async_copy(src_hbm.at[pl.ds(0, chunk)],
                              bpk.at[pl.ds(slot * chunk, chunk)],
                              sems[slot]).wait()
        if vals_hbm is not None:
            pltpu.make_async_copy(vals_hbm.at[pl.ds(0, chunk)],
                                  bv.at[pl.ds(slot * chunk, chunk)],
                                  vsems[slot]).wait()

    start(0, 0)

    def outer(k2, carry):
        for b in range(2):
            k = k2 * 2 + b

            @pl.when(k + 1 < nchunks)
            def _():
                start(k + 1, 1 - b)
            wait(b)
            process(k, b * chunk)
        return carry
    lax.fori_loop(0, nchunks // 2, outer, 0)


def _kg_phase(tbl, w_t, epk, kg_out, src, acc, wsl, bpk, sems,
              cnt=None, cntp=None):
    wid = lax.axis_index("s") * 2 + lax.axis_index("c")
    ones16 = jnp.ones((16,), _f32)
    gpc = ECHUNK // 16

    if cnt is not None:
        _zero(cnt, NE // 16)

    for p in range(NPASS):
        g = wid * NPASS + p
        pltpu.sync_copy(tbl.at[pl.ds(g * (GCH * NE), GCH * NE)], src)
        pltpu.sync_copy(w_t.at[pl.ds(g * (GCH * NREL), GCH * NREL)], wsl)
        _zero(acc, GCH * NE // 16)

        def process(k, off):
            def grp(j, carry2):
                pk = bpk[pl.ds(off + j * 16, 16)]
                h16 = pk & 0x3FFF
                t16 = (pk >> 14) & 0x3FFF
                e16 = pk >> 28
                for c in range(GCH):
                    sv = plsc.load_gather(src, [t16 + c * NE])
                    rv = plsc.load_gather(wsl, [e16 + c * NREL])
                    plsc.addupdate_scatter(acc, [h16 + c * NE], sv * rv)
                if cnt is not None and p == 0:
                    mvec = jnp.broadcast_to(
                        lax.rem(k * gpc + j, NW) == wid, (16,))
                    plsc.addupdate_scatter(cnt, [h16], ones16, mask=mvec)
                return carry2
            lax.fori_loop(0, gpc, grp, 0)

        _stream_chunks(epk, None, NEDGE // ECHUNK, ECHUNK, bpk, None,
                       sems, None, process)
        pltpu.sync_copy(acc, kg_out.at[pl.ds(g * (GCH * NE), GCH * NE)])
    if cnt is not None:
        pltpu.sync_copy(cnt, cntp.at[pl.ds(wid * NE, NE)])


def _spmm_phase(asp_t, rcpk, vals, out, n_rows, nnz, chunk,
                acc, asl, bpk, bv, sems, vsems):
    wid = lax.axis_index("s") * 2 + lax.axis_index("c")

    for p in range(NPASS):
        g = wid * NPASS + p
        pltpu.sync_copy(asp_t.at[pl.ds(g * (GCH * NA), GCH * NA)], asl)
        _zero(acc, GCH * NE // 16)

        def process(k, off):
            def grp(j, carry2):
                pk = bpk[pl.ds(off + j * 16, 16)]
                v16 = bv[pl.ds(off + j * 16, 16)]
                r16 = pk & 0x1FFF
                c16 = pk >> 13
                for c in range(GCH):
                    av = plsc.load_gather(asl, [c16 + c * NA])
                    plsc.addupdate_scatter(acc, [r16 + c * NE], av * v16)
                return carry2
            lax.fori_loop(0, chunk // 16, grp, 0)

        _stream_chunks(rcpk, vals, nnz // chunk, chunk, bpk, bv,
                       sems, vsems, process)

        for c in range(GCH):
            pltpu.sync_copy(acc.at[pl.ds(c * NE, n_rows)],
                            out.at[pl.ds((g * GCH + c) * n_rows, n_rows)])


def _sc1_body(ent_t, w_t, asp_t, epk, ua_pk, ua_v, ia_pk, ia_v,
              kg, cntp, uas, ias,
              src, acc, cnt, wsl, asl, bpk, bv, s0, s1, v0, v1):
    sems = (s0, s1)
    vsems = (v0, v1)
    _kg_phase(ent_t, w_t, epk, kg, src, acc, wsl, bpk, sems,
              cnt=cnt, cntp=cntp)
    _spmm_phase(asp_t, ua_pk, ua_v, uas, NU, NNZ_UA, UCHUNK,
                acc, asl, bpk, bv, sems, vsems)
    _spmm_phase(asp_t, ia_pk, ia_v, ias, NI, NNZ_IA, ICHUNK,
                acc, asl, bpk, bv, sems, vsems)


def _sc2_body(ent_t, w_t, epk, kg, src, acc, wsl, bpk, s0, s1):
    _kg_phase(ent_t, w_t, epk, kg, src, acc, wsl, bpk, (s0, s1))


_sc1 = pl.kernel(
    _sc1_body,
    out_type=(jax.ShapeDtypeStruct((CH * NE,), _f32),
              jax.ShapeDtypeStruct((NW * NE,), _f32),
              jax.ShapeDtypeStruct((CH * NU,), _f32),
              jax.ShapeDtypeStruct((CH * NI,), _f32)),
    mesh=_mesh,
    compiler_params=pltpu.CompilerParams(needs_layout_passes=False),
    scratch_types=(pltpu.VMEM((GCH * NE,), _f32),
                   pltpu.VMEM((GCH * NE,), _f32),
                   pltpu.VMEM((NE,), _f32),
                   pltpu.VMEM((GCH * NREL,), _f32),
                   pltpu.VMEM((GCH * NA,), _f32),
                   pltpu.VMEM((2 * ECHUNK,), _i32),
                   pltpu.VMEM((2 * UCHUNK,), _f32),
                   pltpu.SemaphoreType.DMA,
                   pltpu.SemaphoreType.DMA,
                   pltpu.SemaphoreType.DMA,
                   pltpu.SemaphoreType.DMA))

_sc2 = pl.kernel(
    _sc2_body,
    out_type=jax.ShapeDtypeStruct((CH * NE,), _f32),
    mesh=_mesh,
    compiler_params=pltpu.CompilerParams(needs_layout_passes=False),
    scratch_types=(pltpu.VMEM((GCH * NE,), _f32),
                   pltpu.VMEM((GCH * NE,), _f32),
                   pltpu.VMEM((GCH * NREL,), _f32),
                   pltpu.VMEM((2 * ECHUNK,), _i32),
                   pltpu.SemaphoreType.DMA,
                   pltpu.SemaphoreType.DMA))


def _tc1_body(kg_ref, cntp_ref, e1_ref, inv_ref):
    cnt = jnp.sum(cntp_ref[...], axis=0, keepdims=True)
    inv = 1.0 / jnp.maximum(cnt, 1.0)
    agg = kg_ref[...] * inv
    ss = jnp.sum(agg * agg, axis=0, keepdims=True)
    rn = 1.0 / jnp.maximum(jnp.sqrt(ss), EPS)
    e1_ref[...] = agg * rn
    inv_ref[...] = inv


def _tc1(kg, cntp):
    b = 1024
    return pl.pallas_call(
        _tc1_body,
        grid=(pl.cdiv(NE, b),),
        in_specs=[pl.BlockSpec((CH, b), lambda i: (0, i)),
                  pl.BlockSpec((NW, b), lambda i: (0, i))],
        out_specs=[pl.BlockSpec((CH, b), lambda i: (0, i)),
                   pl.BlockSpec((1, b), lambda i: (0, i))],
        out_shape=[jax.ShapeDtypeStruct((CH, NE), _f32),
                   jax.ShapeDtypeStruct((1, NE), _f32)],
    )(kg, cntp)


def _tc_item_body(kg2_ref, inv_ref, e1_ref, ent0_ref, ias_ref, item_ref,
                  w1_ref, w2_ref, out_ref):
    agg2 = kg2_ref[...] * inv_ref[...]
    ss2 = jnp.sum(agg2 * agg2, axis=0, keepdims=True)
    e2 = agg2 * (1.0 / jnp.maximum(jnp.sqrt(ss2), EPS))
    ia2 = 2.0 * ias_ref[...]
    ssi = jnp.sum(ia2 * ia2, axis=0, keepdims=True)
    i_n = ia2 * (1.0 / jnp.maximum(jnp.sqrt(ssi), EPS))
    ent_res = ent0_ref[...] + jnp.transpose(e1_ref[...]) + jnp.transpose(e2)
    item_pre = item_ref[...] + 2.0 * jnp.transpose(i_n)
    out_ref[...] = w1_ref[0, 0] * ent_res + w2_ref[0, 0] * item_pre


def _tc_item(kg2, inv, e1, ent0, ias, item_emb, w1, w2):
    b = 512
    return pl.pallas_call(
        _tc_item_body,
        grid=(pl.cdiv(NI, b),),
        in_specs=[pl.BlockSpec((CH, b), lambda i: (0, i)),
                  pl.BlockSpec((1, b), lambda i: (0, i)),
                  pl.BlockSpec((CH, b), lambda i: (0, i)),
                  pl.BlockSpec((b, CH), lambda i: (i, 0)),
                  pl.BlockSpec((CH, b), lambda i: (0, i)),
                  pl.BlockSpec((b, CH), lambda i: (i, 0)),
                  pl.BlockSpec((1, 1), lambda i: (0, 0)),
                  pl.BlockSpec((1, 1), lambda i: (0, 0))],
        out_specs=pl.BlockSpec((b, CH), lambda i: (i, 0)),
        out_shape=jax.ShapeDtypeStruct((NI, CH), _f32),
    )(kg2, inv, e1, ent0, ias, item_emb, w1, w2)


def _tc_user_body(uas_ref, user_ref, out_ref):
    a = 2.0 * uas_ref[...]
    ss = jnp.sum(a * a, axis=0, keepdims=True)
    u = a * (1.0 / jnp.maximum(jnp.sqrt(ss), EPS))
    out_ref[...] = user_ref[...] + 2.0 * jnp.transpose(u)


def _tc_user(uas, user_emb):
    b = 512
    return pl.pallas_call(
        _tc_user_body,
        grid=(NU // b,),
        in_specs=[pl.BlockSpec((CH, b), lambda i: (0, i)),
                  pl.BlockSpec((b, CH), lambda i: (i, 0))],
        out_specs=pl.BlockSpec((b, CH), lambda i: (i, 0)),
        out_shape=jax.ShapeDtypeStruct((NU, CH), _f32),
    )(uas, user_emb)


def kernel(user_emb, item_emb, entity_emb, aspect_emb, weight, W1, W2,
           edge_index, edge_type, ua_indices, ua_values, ia_indices,
           ia_values):
    head = edge_index[0]
    tail = edge_index[1]
    et = edge_type - 2
    epk = head | (tail << 14) | (et << 28)
    ua_pk = ua_indices[0] | (ua_indices[1] << 13)
    ia_pk = ia_indices[0] | (ia_indices[1] << 13)
    ent0_t = entity_emb.T.reshape(-1)
    w_t = weight.T.reshape(-1)
    asp_t = aspect_emb.T.reshape(-1)

    kg1, cntp, uas, ias = _sc1(ent0_t, w_t, asp_t, epk,
                               ua_pk, ua_values, ia_pk, ia_values)
    e1_t, inv = _tc1(kg1.reshape(CH, NE), cntp.reshape(NW, NE))
    kg2 = _sc2(e1_t.reshape(-1), w_t, epk)
    item_res = _tc_item(kg2.reshape(CH, NE), inv, e1_t, entity_emb[:NI],
                        ias.reshape(CH, NI), item_emb,
                        W1.reshape(1, 1), W2.reshape(1, 1))
    user_res = _tc_user(uas.reshape(CH, NU), user_emb)
    return (item_res, user_res)


# speedup vs baseline: 4.2406x; 2.6870x over previous
import jax
import jax.numpy as jnp
from jax import lax
from jax.experimental import pallas as pl
from jax.experimental.pallas import tpu as pltpu
from jax.experimental.pallas import tpu_sc as plsc

NU, NI, NE, NA, CH = 8192, 5000, 10000, 128, 256
NEDGE, NNZ_UA, NNZ_IA = 160000, 65536, 40000
NREL = 8
NW = 32
GCH = 4
NGRP = CH // GCH
NPASS = NGRP // NW
ECHUNK = 10000
UCHUNK = 4096
ICHUNK = 4000
EPS = 1e-12

_f32 = jnp.float32
_i32 = jnp.int32
_mesh = plsc.VectorSubcoreMesh(core_axis_name="c", subcore_axis_name="s")


def _zero(ref, n16):
    zeros16 = jnp.zeros((16,), _f32)

    @plsc.parallel_loop(0, n16, 1, unroll=8)
    def _(i):
        ref[pl.ds(i * 16, 16)] = zeros16


def _stream_chunks(src_hbm, vals_hbm, nchunks, chunk, bpk, bv, sems, vsems,
                   process):
    def start(k, slot):
        pltpu.make_async_copy(src_hbm.at[pl.ds(k * chunk, chunk)],
                              bpk.at[pl.ds(slot * chunk, chunk)],
                              sems[slot]).start()
        if vals_hbm is not None:
            pltpu.make_async_copy(vals_hbm.at[pl.ds(k * chunk, chunk)],
                                  bv.at[pl.ds(slot * chunk, chunk)],
                                  vsems[slot]).start()

    def wait(slot):
        pltpu.make_async_copy(src_hbm.at[pl.ds(0, chunk)],
                              bpk.at[pl.ds(slot * chunk, chunk)],
                              sems[slot]).wait()
        if vals_hbm is not None:
            pltpu.make_async_copy(vals_hbm.at[pl.ds(0, chunk)],
                                  bv.at[pl.ds(slot * chunk, chunk)],
                                  vsems[slot]).wait()

    start(0, 0)

    def outer(k2, carry):
        for b in range(2):
            k = k2 * 2 + b

            @pl.when(k + 1 < nchunks)
            def _():
                start(k + 1, 1 - b)
            wait(b)
            process(k, b * chunk)
        return carry
    lax.fori_loop(0, nchunks // 2, outer, 0)


def _kg_phase(tbl, w_t, epk, kg_out, src, acc, wsl, bpk, sems,
              cnt=None, cntp=None):
    wid = lax.axis_index("s") * 2 + lax.axis_index("c")
    ones16 = jnp.ones((16,), _f32)
    gpc = ECHUNK // 16

    if cnt is not None:
        _zero(cnt, NE // 16)

    for p in range(NPASS):
        g = wid * NPASS + p
        pltpu.sync_copy(tbl.at[pl.ds(g * (GCH * NE), GCH * NE)], src)
        pltpu.sync_copy(w_t.at[pl.ds(g * (GCH * NREL), GCH * NREL)], wsl)
        _zero(acc, GCH * NE // 16)

        def process(k, off):
            @plsc.parallel_loop(0, gpc, 1, unroll=4)
            def _(j):
                pk = bpk[pl.ds(off + j * 16, 16)]
                h16 = pk & 0x3FFF
                t16 = (pk >> 14) & 0x3FFF
                e16 = pk >> 28
                for c in range(GCH):
                    sv = plsc.load_gather(src, [t16 + c * NE])
                    rv = plsc.load_gather(wsl, [e16 + c * NREL])
                    plsc.addupdate_scatter(acc, [h16 + c * NE], sv * rv)
                if cnt is not None and p == 0:
                    mvec = jnp.broadcast_to(
                        lax.rem(k * gpc + j, NW) == wid, (16,))
                    plsc.addupdate_scatter(cnt, [h16], ones16, mask=mvec)

        _stream_chunks(epk, None, NEDGE // ECHUNK, ECHUNK, bpk, None,
                       sems, None, process)
        pltpu.sync_copy(acc, kg_out.at[pl.ds(g * (GCH * NE), GCH * NE)])
    if cnt is not None:
        pltpu.sync_copy(cnt, cntp.at[pl.ds(wid * NE, NE)])


def _spmm_phase(asp_t, rcpk, vals, out, n_rows, nnz, chunk,
                acc, asl, bpk, bv, sems, vsems):
    wid = lax.axis_index("s") * 2 + lax.axis_index("c")

    for p in range(NPASS):
        g = wid * NPASS + p
        pltpu.sync_copy(asp_t.at[pl.ds(g * (GCH * NA), GCH * NA)], asl)
        _zero(acc, GCH * NE // 16)

        def process(k, off):
            @plsc.parallel_loop(0, chunk // 16, 1, unroll=4)
            def _(j):
                pk = bpk[pl.ds(off + j * 16, 16)]
                v16 = bv[pl.ds(off + j * 16, 16)]
                r16 = pk & 0x1FFF
                c16 = pk >> 13
                for c in range(GCH):
                    av = plsc.load_gather(asl, [c16 + c * NA])
                    plsc.addupdate_scatter(acc, [r16 + c * NE], av * v16)

        _stream_chunks(rcpk, vals, nnz // chunk, chunk, bpk, bv,
                       sems, vsems, process)

        for c in range(GCH):
            pltpu.sync_copy(acc.at[pl.ds(c * NE, n_rows)],
                            out.at[pl.ds((g * GCH + c) * n_rows, n_rows)])


def _sc1_body(ent_t, w_t, asp_t, epk, ua_pk, ua_v, ia_pk, ia_v,
              kg, cntp, uas, ias,
              src, acc, cnt, wsl, asl, bpk, bv, s0, s1, v0, v1):
    sems = (s0, s1)
    vsems = (v0, v1)
    _kg_phase(ent_t, w_t, epk, kg, src, acc, wsl, bpk, sems,
              cnt=cnt, cntp=cntp)
    _spmm_phase(asp_t, ua_pk, ua_v, uas, NU, NNZ_UA, UCHUNK,
                acc, asl, bpk, bv, sems, vsems)
    _spmm_phase(asp_t, ia_pk, ia_v, ias, NI, NNZ_IA, ICHUNK,
                acc, asl, bpk, bv, sems, vsems)


def _sc2_body(ent_t, w_t, epk, kg, src, acc, wsl, bpk, s0, s1):
    _kg_phase(ent_t, w_t, epk, kg, src, acc, wsl, bpk, (s0, s1))


_sc1 = pl.kernel(
    _sc1_body,
    out_type=(jax.ShapeDtypeStruct((CH * NE,), _f32),
              jax.ShapeDtypeStruct((NW * NE,), _f32),
              jax.ShapeDtypeStruct((CH * NU,), _f32),
              jax.ShapeDtypeStruct((CH * NI,), _f32)),
    mesh=_mesh,
    compiler_params=pltpu.CompilerParams(needs_layout_passes=False),
    scratch_types=(pltpu.VMEM((GCH * NE,), _f32),
                   pltpu.VMEM((GCH * NE,), _f32),
                   pltpu.VMEM((NE,), _f32),
                   pltpu.VMEM((GCH * NREL,), _f32),
                   pltpu.VMEM((GCH * NA,), _f32),
                   pltpu.VMEM((2 * ECHUNK,), _i32),
                   pltpu.VMEM((2 * UCHUNK,), _f32),
                   pltpu.SemaphoreType.DMA,
                   pltpu.SemaphoreType.DMA,
                   pltpu.SemaphoreType.DMA,
                   pltpu.SemaphoreType.DMA))

_sc2 = pl.kernel(
    _sc2_body,
    out_type=jax.ShapeDtypeStruct((CH * NE,), _f32),
    mesh=_mesh,
    compiler_params=pltpu.CompilerParams(needs_layout_passes=False),
    scratch_types=(pltpu.VMEM((GCH * NE,), _f32),
                   pltpu.VMEM((GCH * NE,), _f32),
                   pltpu.VMEM((GCH * NREL,), _f32),
                   pltpu.VMEM((2 * ECHUNK,), _i32),
                   pltpu.SemaphoreType.DMA,
                   pltpu.SemaphoreType.DMA))


def _tc1_body(kg_ref, cntp_ref, e1_ref, inv_ref):
    cnt = jnp.sum(cntp_ref[...], axis=0, keepdims=True)
    inv = 1.0 / jnp.maximum(cnt, 1.0)
    agg = kg_ref[...] * inv
    ss = jnp.sum(agg * agg, axis=0, keepdims=True)
    rn = 1.0 / jnp.maximum(jnp.sqrt(ss), EPS)
    e1_ref[...] = agg * rn
    inv_ref[...] = inv


def _tc1(kg, cntp):
    b = 1024
    return pl.pallas_call(
        _tc1_body,
        grid=(pl.cdiv(NE, b),),
        in_specs=[pl.BlockSpec((CH, b), lambda i: (0, i)),
                  pl.BlockSpec((NW, b), lambda i: (0, i))],
        out_specs=[pl.BlockSpec((CH, b), lambda i: (0, i)),
                   pl.BlockSpec((1, b), lambda i: (0, i))],
        out_shape=[jax.ShapeDtypeStruct((CH, NE), _f32),
                   jax.ShapeDtypeStruct((1, NE), _f32)],
    )(kg, cntp)


def _tc_item_body(kg2_ref, inv_ref, e1_ref, ent0_ref, ias_ref, item_ref,
                  w1_ref, w2_ref, out_ref):
    agg2 = kg2_ref[...] * inv_ref[...]
    ss2 = jnp.sum(agg2 * agg2, axis=0, keepdims=True)
    e2 = agg2 * (1.0 / jnp.maximum(jnp.sqrt(ss2), EPS))
    ia2 = 2.0 * ias_ref[...]
    ssi = jnp.sum(ia2 * ia2, axis=0, keepdims=True)
    i_n = ia2 * (1.0 / jnp.maximum(jnp.sqrt(ssi), EPS))
    ent_res = ent0_ref[...] + jnp.transpose(e1_ref[...]) + jnp.transpose(e2)
    item_pre = item_ref[...] + 2.0 * jnp.transpose(i_n)
    out_ref[...] = w1_ref[0, 0] * ent_res + w2_ref[0, 0] * item_pre


def _tc_item(kg2, inv, e1, ent0, ias, item_emb, w1, w2):
    b = 512
    return pl.pallas_call(
        _tc_item_body,
        grid=(pl.cdiv(NI, b),),
        in_specs=[pl.BlockSpec((CH, b), lambda i: (0, i)),
                  pl.BlockSpec((1, b), lambda i: (0, i)),
                  pl.BlockSpec((CH, b), lambda i: (0, i)),
                  pl.BlockSpec((b, CH), lambda i: (i, 0)),
                  pl.BlockSpec((CH, b), lambda i: (0, i)),
                  pl.BlockSpec((b, CH), lambda i: (i, 0)),
                  pl.BlockSpec((1, 1), lambda i: (0, 0)),
                  pl.BlockSpec((1, 1), lambda i: (0, 0))],
        out_specs=pl.BlockSpec((b, CH), lambda i: (i, 0)),
        out_shape=jax.ShapeDtypeStruct((NI, CH), _f32),
    )(kg2, inv, e1, ent0, ias, item_emb, w1, w2)


def _tc_user_body(uas_ref, user_ref, out_ref):
    a = 2.0 * uas_ref[...]
    ss = jnp.sum(a * a, axis=0, keepdims=True)
    u = a * (1.0 / jnp.maximum(jnp.sqrt(ss), EPS))
    out_ref[...] = user_ref[...] + 2.0 * jnp.transpose(u)


def _tc_user(uas, user_emb):
    b = 512
    return pl.pallas_call(
        _tc_user_body,
        grid=(NU // b,),
        in_specs=[pl.BlockSpec((CH, b), lambda i: (0, i)),
                  pl.BlockSpec((b, CH), lambda i: (i, 0))],
        out_specs=pl.BlockSpec((b, CH), lambda i: (i, 0)),
        out_shape=jax.ShapeDtypeStruct((NU, CH), _f32),
    )(uas, user_emb)


def kernel(user_emb, item_emb, entity_emb, aspect_emb, weight, W1, W2,
           edge_index, edge_type, ua_indices, ua_values, ia_indices,
           ia_values):
    head = edge_index[0]
    tail = edge_index[1]
    et = edge_type - 2
    epk = head | (tail << 14) | (et << 28)
    ua_pk = ua_indices[0] | (ua_indices[1] << 13)
    ia_pk = ia_indices[0] | (ia_indices[1] << 13)
    ent0_t = entity_emb.T.reshape(-1)
    w_t = weight.T.reshape(-1)
    asp_t = aspect_emb.T.reshape(-1)

    kg1, cntp, uas, ias = _sc1(ent0_t, w_t, asp_t, epk,
                               ua_pk, ua_values, ia_pk, ia_values)
    e1_t, inv = _tc1(kg1.reshape(CH, NE), cntp.reshape(NW, NE))
    kg2 = _sc2(e1_t.reshape(-1), w_t, epk)
    item_res = _tc_item(kg2.reshape(CH, NE), inv, e1_t, entity_emb[:NI],
                        ias.reshape(CH, NI), item_emb,
                        W1.reshape(1, 1), W2.reshape(1, 1))
    user_res = _tc_user(uas.reshape(CH, NU), user_emb)
    return (item_res, user_res)
